# baseline (speedup 1.0000x reference)
import math

import jax
import jax.numpy as jnp
from jax import lax
from jax.experimental import pallas as pl
from jax.experimental.pallas import tpu as pltpu

_SCORE_THRESH = 0.05
_NMS_THRESH = 0.5
_DET_PER_IMG = 100
_MIN_SIZE = 1e-2
_BBOX_XFORM_CLIP = math.log(1000.0 / 16)
_IMG_H = 800.0
_IMG_W = 800.0
_N = 20000
_ROWS = 157
_PAD = _ROWS * 128


def _nms_body(l0, l1, dx, dy, dw, dh, px1, py1, px2, py2,
              coords_ref, score_ref,
              sm_ref, x1_ref, y1_ref, x2_ref, y2_ref, s_ref,
              kx_ref, ky_ref, kx2_ref, ky2_ref, ka_ref):
    neg_inf = jnp.float32(-jnp.inf)

    row = lax.broadcasted_iota(jnp.int32, (_ROWS, 128), 0)
    col = lax.broadcasted_iota(jnp.int32, (_ROWS, 128), 1)
    idx = row * 128 + col
    inb = idx < _N

    m = jnp.maximum(l0[...], l1[...])
    e0 = jnp.exp(l0[...] - m)
    e1 = jnp.exp(l1[...] - m)
    s = e1 / (e0 + e1)

    w = px2[...] - px1[...]
    h = py2[...] - py1[...]
    cx = px1[...] + 0.5 * w
    cy = py1[...] + 0.5 * h
    tx = dx[...] / 10.0
    ty = dy[...] / 10.0
    tw = jnp.minimum(dw[...] / 5.0, _BBOX_XFORM_CLIP)
    th = jnp.minimum(dh[...] / 5.0, _BBOX_XFORM_CLIP)
    pcx = tx * w + cx
    pcy = ty * h + cy
    pw = jnp.exp(tw) * w
    ph = jnp.exp(th) * h
    x1 = pcx - 0.5 * pw
    y1 = pcy - 0.5 * ph
    x2 = pcx + 0.5 * pw
    y2 = pcy + 0.5 * ph
    x1 = jnp.clip(x1, 0.0, _IMG_W)
    x2 = jnp.clip(x2, 0.0, _IMG_W)
    y1 = jnp.clip(y1, 0.0, _IMG_H)
    y2 = jnp.clip(y2, 0.0, _IMG_H)

    ws = x2 - x1
    hs = y2 - y1
    valid = (s > _SCORE_THRESH) & (ws >= _MIN_SIZE) & (hs >= _MIN_SIZE) & inb

    coord_max = jnp.maximum(jnp.maximum(x1, y1), jnp.maximum(x2, y2))
    mc = jnp.max(jnp.where(valid, coord_max, neg_inf))
    off = mc + 1.0

    sm_ref[...] = jnp.where(valid, s, neg_inf)
    x1_ref[...] = x1
    y1_ref[...] = y1
    x2_ref[...] = x2
    y2_ref[...] = y2
    s_ref[...] = s

    zrow = jnp.zeros((1, 128), jnp.float32)
    coords_ref[...] = jnp.zeros((4, 128), jnp.float32)
    score_ref[...] = zrow
    kx_ref[...] = zrow
    ky_ref[...] = zrow
    kx2_ref[...] = zrow
    ky2_ref[...] = zrow
    ka_ref[...] = zrow

    big = jnp.int32(2**30)
    lane = lax.broadcasted_iota(jnp.int32, (1, 128), 1)

    def cond(t):
        return t < _DET_PER_IMG

    def body(t):
        sm = sm_ref[...]
        mx = jnp.max(sm)
        i = jnp.min(jnp.where(sm == mx, idx, big))
        r = i // 128
        c = i - r * 128
        lm = lane == c
        xi1 = jnp.sum(jnp.where(lm, x1_ref[pl.ds(r, 1), :], 0.0))
        yi1 = jnp.sum(jnp.where(lm, y1_ref[pl.ds(r, 1), :], 0.0))
        xi2 = jnp.sum(jnp.where(lm, x2_ref[pl.ds(r, 1), :], 0.0))
        yi2 = jnp.sum(jnp.where(lm, y2_ref[pl.ds(r, 1), :], 0.0))
        si = jnp.sum(jnp.where(lm, s_ref[pl.ds(r, 1), :], 0.0))
        nxi1 = xi1 + off
        nyi1 = yi1 + off
        nxi2 = xi2 + off
        nyi2 = yi2 + off
        area_i = (nxi2 - nxi1) * (nyi2 - nyi1)
        xx1 = jnp.maximum(nxi1, kx_ref[...])
        yy1 = jnp.maximum(nyi1, ky_ref[...])
        xx2 = jnp.minimum(nxi2, kx2_ref[...])
        yy2 = jnp.minimum(nyi2, ky2_ref[...])
        iw = jnp.maximum(xx2 - xx1, 0.0)
        ih = jnp.maximum(yy2 - yy1, 0.0)
        inter = iw * ih
        iou = inter / (area_i + ka_ref[...] - inter)
        conflict = jnp.max(iou) > _NMS_THRESH
        accept = jnp.logical_or(jnp.logical_not(conflict), mx == neg_inf)
        sm_ref[pl.ds(r, 1), :] = jnp.where(lm, neg_inf, sm_ref[pl.ds(r, 1), :])
        tm = (lane == t) & accept
        coords_ref[0:1, :] = jnp.where(tm, xi1, coords_ref[0:1, :])
        coords_ref[1:2, :] = jnp.where(tm, yi1, coords_ref[1:2, :])
        coords_ref[2:3, :] = jnp.where(tm, xi2, coords_ref[2:3, :])
        coords_ref[3:4, :] = jnp.where(tm, yi2, coords_ref[3:4, :])
        score_ref[...] = jnp.where(tm, si, score_ref[...])
        kx_ref[...] = jnp.where(tm, nxi1, kx_ref[...])
        ky_ref[...] = jnp.where(tm, nyi1, ky_ref[...])
        kx2_ref[...] = jnp.where(tm, nxi2, kx2_ref[...])
        ky2_ref[...] = jnp.where(tm, nyi2, ky2_ref[...])
        ka_ref[...] = jnp.where(tm, area_i, ka_ref[...])
        return t + accept.astype(jnp.int32)

    lax.while_loop(cond, body, jnp.int32(0))


def _pad2d(v):
    return jnp.pad(v, (0, _PAD - _N)).reshape(_ROWS, 128)


def kernel(class_logits, box_regression, proposals):
    args = [
        class_logits[:, 0], class_logits[:, 1],
        box_regression[:, 4], box_regression[:, 5],
        box_regression[:, 6], box_regression[:, 7],
        proposals[:, 0], proposals[:, 1],
        proposals[:, 2], proposals[:, 3],
    ]
    args = [_pad2d(a) for a in args]
    coords, score = pl.pallas_call(
        _nms_body,
        out_shape=(
            jax.ShapeDtypeStruct((4, 128), jnp.float32),
            jax.ShapeDtypeStruct((1, 128), jnp.float32),
        ),
        scratch_shapes=[pltpu.VMEM((_ROWS, 128), jnp.float32)] * 6
        + [pltpu.VMEM((1, 128), jnp.float32)] * 5,
    )(*args)
    boxes = coords[:, :_DET_PER_IMG].T
    scores = score[0, :_DET_PER_IMG]
    labels = jnp.ones((_DET_PER_IMG,), jnp.int32)
    return boxes, scores, labels


# speedup vs baseline: 1.1396x; 1.1396x over previous
import math

import jax
import jax.numpy as jnp
from jax import lax
from jax.experimental import pallas as pl
from jax.experimental.pallas import tpu as pltpu

_SCORE_THRESH = 0.05
_NMS_THRESH = 0.5
_DET_PER_IMG = 100
_MIN_SIZE = 1e-2
_BBOX_XFORM_CLIP = math.log(1000.0 / 16)
_IMG_H = 800.0
_IMG_W = 800.0
_N = 20000
_ROWS = 160
_PAD = _ROWS * 128


def _nms_body(l0, l1, dx, dy, dw, dh, px1, py1, px2, py2,
              coords_ref, score_ref,
              sm_ref, x1_ref, y1_ref, x2_ref, y2_ref, s_ref,
              kx_ref, ky_ref, kx2_ref, ky2_ref, ka_ref):
    neg_inf = jnp.float32(-jnp.inf)

    row = lax.broadcasted_iota(jnp.int32, (_ROWS, 128), 0)
    col = lax.broadcasted_iota(jnp.int32, (_ROWS, 128), 1)
    idx = row * 128 + col
    inb = idx < _N

    m = jnp.maximum(l0[...], l1[...])
    e0 = jnp.exp(l0[...] - m)
    e1 = jnp.exp(l1[...] - m)
    s = e1 / (e0 + e1)

    w = px2[...] - px1[...]
    h = py2[...] - py1[...]
    cx = px1[...] + 0.5 * w
    cy = py1[...] + 0.5 * h
    tx = dx[...] / 10.0
    ty = dy[...] / 10.0
    tw = jnp.minimum(dw[...] / 5.0, _BBOX_XFORM_CLIP)
    th = jnp.minimum(dh[...] / 5.0, _BBOX_XFORM_CLIP)
    pcx = tx * w + cx
    pcy = ty * h + cy
    pw = jnp.exp(tw) * w
    ph = jnp.exp(th) * h
    x1 = pcx - 0.5 * pw
    y1 = pcy - 0.5 * ph
    x2 = pcx + 0.5 * pw
    y2 = pcy + 0.5 * ph
    x1 = jnp.clip(x1, 0.0, _IMG_W)
    x2 = jnp.clip(x2, 0.0, _IMG_W)
    y1 = jnp.clip(y1, 0.0, _IMG_H)
    y2 = jnp.clip(y2, 0.0, _IMG_H)

    ws = x2 - x1
    hs = y2 - y1
    valid = (s > _SCORE_THRESH) & (ws >= _MIN_SIZE) & (hs >= _MIN_SIZE) & inb

    coord_max = jnp.maximum(jnp.maximum(x1, y1), jnp.maximum(x2, y2))
    mc = jnp.max(jnp.where(valid, coord_max, neg_inf))
    off = mc + 1.0

    sm_ref[...] = jnp.where(valid, s, neg_inf)
    x1_ref[...] = x1
    y1_ref[...] = y1
    x2_ref[...] = x2
    y2_ref[...] = y2
    s_ref[...] = s

    zrow = jnp.zeros((1, 128), jnp.float32)
    coords_ref[...] = jnp.zeros((4, 128), jnp.float32)
    score_ref[...] = zrow
    kx_ref[...] = zrow
    ky_ref[...] = zrow
    kx2_ref[...] = zrow
    ky2_ref[...] = zrow
    ka_ref[...] = zrow

    big = jnp.int32(2**30)
    lane = lax.broadcasted_iota(jnp.int32, (1, 128), 1)

    def cond(t):
        return t < _DET_PER_IMG

    def body(t):
        sm = sm_ref[...]
        colmax = jnp.max(sm.reshape(_ROWS // 8, 8, 128), axis=(0, 1),
                         keepdims=True).reshape(1, 128)
        cand = jnp.where(sm == colmax, idx, big)
        colmin = jnp.min(cand.reshape(_ROWS // 8, 8, 128), axis=(0, 1),
                         keepdims=True).reshape(1, 128)
        mxv = jnp.max(colmax, axis=1, keepdims=True)
        i = jnp.min(jnp.where(colmax == mxv, colmin, big))
        exhausted = mxv[0, 0] == neg_inf
        r = i // 128
        c = i - r * 128
        lm = lane == c
        stacked = jnp.concatenate(
            [x1_ref[pl.ds(r, 1), :], y1_ref[pl.ds(r, 1), :],
             x2_ref[pl.ds(r, 1), :], y2_ref[pl.ds(r, 1), :],
             s_ref[pl.ds(r, 1), :]], axis=0)
        vals = jnp.sum(jnp.where(lane == c, stacked, 0.0), axis=1,
                       keepdims=True)
        bc = jnp.broadcast_to(vals, (5, 128))
        xi1 = bc[0:1, :]
        yi1 = bc[1:2, :]
        xi2 = bc[2:3, :]
        yi2 = bc[3:4, :]
        si = bc[4:5, :]
        nxi1 = xi1 + off
        nyi1 = yi1 + off
        nxi2 = xi2 + off
        nyi2 = yi2 + off
        area_i = (nxi2 - nxi1) * (nyi2 - nyi1)
        xx1 = jnp.maximum(nxi1, kx_ref[...])
        yy1 = jnp.maximum(nyi1, ky_ref[...])
        xx2 = jnp.minimum(nxi2, kx2_ref[...])
        yy2 = jnp.minimum(nyi2, ky2_ref[...])
        iw = jnp.maximum(xx2 - xx1, 0.0)
        ih = jnp.maximum(yy2 - yy1, 0.0)
        inter = iw * ih
        iou = inter / (area_i + ka_ref[...] - inter)
        conflict = jnp.max(iou) > _NMS_THRESH
        accept = jnp.logical_or(jnp.logical_not(conflict), exhausted)
        sm_ref[pl.ds(r, 1), :] = jnp.where(lm, neg_inf, sm_ref[pl.ds(r, 1), :])
        tm = (lane == t) & accept
        coords_ref[0:1, :] = jnp.where(tm, xi1, coords_ref[0:1, :])
        coords_ref[1:2, :] = jnp.where(tm, yi1, coords_ref[1:2, :])
        coords_ref[2:3, :] = jnp.where(tm, xi2, coords_ref[2:3, :])
        coords_ref[3:4, :] = jnp.where(tm, yi2, coords_ref[3:4, :])
        score_ref[...] = jnp.where(tm, si, score_ref[...])
        kx_ref[...] = jnp.where(tm, nxi1, kx_ref[...])
        ky_ref[...] = jnp.where(tm, nyi1, ky_ref[...])
        kx2_ref[...] = jnp.where(tm, nxi2, kx2_ref[...])
        ky2_ref[...] = jnp.where(tm, nyi2, ky2_ref[...])
        ka_ref[...] = jnp.where(tm, area_i, ka_ref[...])
        return t + accept.astype(jnp.int32)

    lax.while_loop(cond, body, jnp.int32(0))


def _pad2d(v):
    return jnp.pad(v, (0, _PAD - _N)).reshape(_ROWS, 128)


def kernel(class_logits, box_regression, proposals):
    args = [
        class_logits[:, 0], class_logits[:, 1],
        box_regression[:, 4], box_regression[:, 5],
        box_regression[:, 6], box_regression[:, 7],
        proposals[:, 0], proposals[:, 1],
        proposals[:, 2], proposals[:, 3],
    ]
    args = [_pad2d(a) for a in args]
    coords, score = pl.pallas_call(
        _nms_body,
        out_shape=(
            jax.ShapeDtypeStruct((4, 128), jnp.float32),
            jax.ShapeDtypeStruct((1, 128), jnp.float32),
        ),
        scratch_shapes=[pltpu.VMEM((_ROWS, 128), jnp.float32)] * 6
        + [pltpu.VMEM((1, 128), jnp.float32)] * 5,
    )(*args)
    boxes = coords[:, :_DET_PER_IMG].T
    scores = score[0, :_DET_PER_IMG]
    labels = jnp.ones((_DET_PER_IMG,), jnp.int32)
    return boxes, scores, labels


# speedup vs baseline: 2.3102x; 2.0273x over previous
import math

import jax
import jax.numpy as jnp
from jax import lax
from jax.experimental import pallas as pl
from jax.experimental.pallas import tpu as pltpu

_SCORE_THRESH = 0.05
_NMS_THRESH = 0.5
_DET_PER_IMG = 100
_MIN_SIZE = 1e-2
_BBOX_XFORM_CLIP = math.log(1000.0 / 16)
_IMG_H = 800.0
_IMG_W = 800.0
_N = 20000
_ROWS = 160
_PAD = _ROWS * 128
_TILES = _ROWS // 8


def _nms_body(l0, l1, dx, dy, dw, dh, px1, py1, px2, py2,
              coords_ref, score_ref,
              sm_ref, x1_ref, y1_ref, x2_ref, y2_ref, s_ref,
              idxf_ref, rowf_ref, p1_ref, p2_ref,
              kx_ref, ky_ref, kx2_ref, ky2_ref, ka_ref):
    neg_inf = jnp.float32(-jnp.inf)
    fbig = jnp.float32(2**30)

    row = lax.broadcasted_iota(jnp.int32, (_ROWS, 128), 0)
    col = lax.broadcasted_iota(jnp.int32, (_ROWS, 128), 1)
    idx = row * 128 + col
    inb = idx < _N

    m = jnp.maximum(l0[...], l1[...])
    e0 = jnp.exp(l0[...] - m)
    e1 = jnp.exp(l1[...] - m)
    s = e1 / (e0 + e1)

    w = px2[...] - px1[...]
    h = py2[...] - py1[...]
    cx = px1[...] + 0.5 * w
    cy = py1[...] + 0.5 * h
    tx = dx[...] / 10.0
    ty = dy[...] / 10.0
    tw = jnp.minimum(dw[...] / 5.0, _BBOX_XFORM_CLIP)
    th = jnp.minimum(dh[...] / 5.0, _BBOX_XFORM_CLIP)
    pcx = tx * w + cx
    pcy = ty * h + cy
    pw = jnp.exp(tw) * w
    ph = jnp.exp(th) * h
    x1 = pcx - 0.5 * pw
    y1 = pcy - 0.5 * ph
    x2 = pcx + 0.5 * pw
    y2 = pcy + 0.5 * ph
    x1 = jnp.clip(x1, 0.0, _IMG_W)
    x2 = jnp.clip(x2, 0.0, _IMG_W)
    y1 = jnp.clip(y1, 0.0, _IMG_H)
    y2 = jnp.clip(y2, 0.0, _IMG_H)

    ws = x2 - x1
    hs = y2 - y1
    valid = (s > _SCORE_THRESH) & (ws >= _MIN_SIZE) & (hs >= _MIN_SIZE) & inb

    coord_max = jnp.maximum(jnp.maximum(x1, y1), jnp.maximum(x2, y2))
    mc = jnp.max(jnp.where(valid, coord_max, neg_inf))
    off = mc + 1.0

    sm_ref[...] = jnp.where(valid, s, neg_inf)
    x1_ref[...] = x1
    y1_ref[...] = y1
    x2_ref[...] = x2
    y2_ref[...] = y2
    s_ref[...] = s
    idxf_ref[...] = idx.astype(jnp.float32)
    rowf_ref[...] = row.astype(jnp.float32)

    zrow = jnp.zeros((1, 128), jnp.float32)
    coords_ref[...] = jnp.zeros((4, 128), jnp.float32)
    score_ref[...] = zrow
    p1_ref[...] = jnp.zeros((2, 128), jnp.float32)
    p2_ref[...] = jnp.zeros((6, 128), jnp.float32)
    kx_ref[...] = zrow
    ky_ref[...] = zrow
    kx2_ref[...] = zrow
    ky2_ref[...] = zrow
    ka_ref[...] = zrow

    lane = lax.broadcasted_iota(jnp.int32, (1, 128), 1)

    def cond(state):
        t, _ = state
        return t < _DET_PER_IMG

    def body(state):
        t, k = state

        bx1 = p2_ref[0:1, :]
        by1 = p2_ref[1:2, :]
        bx2 = p2_ref[2:3, :]
        by2 = p2_ref[3:4, :]
        bs = p2_ref[4:5, :]
        bexh = p2_ref[5:6, :]
        nbx1 = bx1 + off
        nby1 = by1 + off
        nbx2 = bx2 + off
        nby2 = by2 + off
        area_b = (nbx2 - nbx1) * (nby2 - nby1)
        xx1 = jnp.maximum(nbx1, kx_ref[...])
        yy1 = jnp.maximum(nby1, ky_ref[...])
        xx2 = jnp.minimum(nbx2, kx2_ref[...])
        yy2 = jnp.minimum(nby2, ky2_ref[...])
        iw = jnp.maximum(xx2 - xx1, 0.0)
        ih = jnp.maximum(yy2 - yy1, 0.0)
        inter = iw * ih
        iou = inter / (area_b + ka_ref[...] - inter)
        ioum = jnp.where(bexh > 0.0, -1.0, iou)
        cmax = jnp.max(ioum)
        accept = jnp.logical_and(cmax <= _NMS_THRESH, k >= 2)
        tm = (lane == t) & accept
        coords_ref[0:1, :] = jnp.where(tm, bx1, coords_ref[0:1, :])
        coords_ref[1:2, :] = jnp.where(tm, by1, coords_ref[1:2, :])
        coords_ref[2:3, :] = jnp.where(tm, bx2, coords_ref[2:3, :])
        coords_ref[3:4, :] = jnp.where(tm, by2, coords_ref[3:4, :])
        score_ref[...] = jnp.where(tm, bs, score_ref[...])
        kx_ref[...] = jnp.where(tm, nbx1, kx_ref[...])
        ky_ref[...] = jnp.where(tm, nby1, ky_ref[...])
        kx2_ref[...] = jnp.where(tm, nbx2, kx2_ref[...])
        ky2_ref[...] = jnp.where(tm, nby2, ky2_ref[...])
        ka_ref[...] = jnp.where(tm, area_b, ka_ref[...])
        t2 = t + accept.astype(jnp.int32)

        iv1 = p1_ref[0:1, :]
        exh1 = p1_ref[1:2, :]
        rv = jnp.floor(iv1 / 128.0)
        cv = iv1 - rv * 128.0
        rowm = rowf_ref[...] == rv

        def _gather_row(ref):
            g = jnp.where(rowm, ref[...], 0.0)
            return jnp.sum(g.reshape(_TILES, 8, 128), axis=(0, 1),
                           keepdims=True).reshape(1, 128)

        stacked = jnp.concatenate(
            [_gather_row(x1_ref), _gather_row(y1_ref),
             _gather_row(x2_ref), _gather_row(y2_ref),
             _gather_row(s_ref)], axis=0)
        lanef = lane.astype(jnp.float32)
        ext = jnp.sum(jnp.where(lanef == cv, stacked, 0.0), axis=1,
                      keepdims=True)
        p2_ref[0:5, :] = jnp.broadcast_to(ext, (5, 128))
        p2_ref[5:6, :] = exh1

        sm = sm_ref[...]
        colmax = jnp.max(sm.reshape(_TILES, 8, 128), axis=(0, 1),
                         keepdims=True).reshape(1, 128)
        cand = jnp.where(sm == colmax, idxf_ref[...], fbig)
        colmin = jnp.min(cand.reshape(_TILES, 8, 128), axis=(0, 1),
                         keepdims=True).reshape(1, 128)
        mxv = jnp.max(colmax, axis=1, keepdims=True)
        iv = jnp.min(jnp.where(colmax == mxv, colmin, fbig), axis=1,
                     keepdims=True)
        exhf = jnp.where(mxv == neg_inf, 1.0, 0.0)
        sm_ref[...] = jnp.where(idxf_ref[...] == iv, neg_inf, sm)
        p1_ref[0:1, :] = jnp.broadcast_to(iv, (1, 128))
        p1_ref[1:2, :] = jnp.broadcast_to(exhf, (1, 128))

        return t2, k + 1

    lax.while_loop(cond, body, (jnp.int32(0), jnp.int32(0)))


def _pad2d(v):
    return jnp.pad(v, (0, _PAD - _N)).reshape(_ROWS, 128)


def kernel(class_logits, box_regression, proposals):
    args = [
        class_logits[:, 0], class_logits[:, 1],
        box_regression[:, 4], box_regression[:, 5],
        box_regression[:, 6], box_regression[:, 7],
        proposals[:, 0], proposals[:, 1],
        proposals[:, 2], proposals[:, 3],
    ]
    args = [_pad2d(a) for a in args]
    coords, score = pl.pallas_call(
        _nms_body,
        out_shape=(
            jax.ShapeDtypeStruct((4, 128), jnp.float32),
            jax.ShapeDtypeStruct((1, 128), jnp.float32),
        ),
        scratch_shapes=[pltpu.VMEM((_ROWS, 128), jnp.float32)] * 8
        + [pltpu.VMEM((2, 128), jnp.float32),
           pltpu.VMEM((6, 128), jnp.float32)]
        + [pltpu.VMEM((1, 128), jnp.float32)] * 5,
    )(*args)
    boxes = coords[:, :_DET_PER_IMG].T
    scores = score[0, :_DET_PER_IMG]
    labels = jnp.ones((_DET_PER_IMG,), jnp.int32)
    return boxes, scores, labels
